# baseline (speedup 1.0000x reference)
import jax
import jax.numpy as jnp
from jax import lax
from jax.experimental import pallas as pl
from jax.experimental.pallas import tpu as pltpu
from jax.experimental.pallas import tpu_sc as plsc

NC = 2
NS = 16
LANES = 16
NW = NC * NS

B, L, V, D = 4096, 200, 1000000, 64
DP = 128
N = B * L
N_PER_W = N // NW
CHUNK = 128
NBUF = 5
N_CHUNKS = N_PER_W // CHUNK
LOOKAHEAD = 4
STEADY = (N_CHUNKS - 1 - LOOKAHEAD) // NBUF


def _body(x_hbm, tok_hbm, pos_hbm, out_hbm, *refs):
  idx = refs[0:NBUF]
  idx2 = refs[NBUF:2 * NBUF]
  tok = refs[2 * NBUF:3 * NBUF]
  pos_l = refs[3 * NBUF]
  gsem = refs[3 * NBUF + 1:3 * NBUF + 1 + NBUF]
  wsem = refs[3 * NBUF + 1 + NBUF:3 * NBUF + 1 + 2 * NBUF]

  wid = lax.axis_index("s") * NC + lax.axis_index("c")
  w_base = wid * N_PER_W

  pltpu.sync_copy(pos_hbm, pos_l)

  def fire_gather(c, k):
    base = w_base + c * CHUNK
    pltpu.sync_copy(x_hbm.at[pl.ds(base, CHUNK)], idx[k])

    def pidx_body(g, _):
      xv = idx[k][pl.ds(g * LANES, LANES)]
      idx2[k][pl.ds(g * LANES, LANES)] = xv * 2
      return 0
    lax.fori_loop(0, CHUNK // LANES, pidx_body, 0)
    pltpu.async_copy(tok_hbm.at[idx2[k]], tok[k], gsem[k])

  def wait_gather(k):
    pltpu.make_async_copy(tok_hbm.at[idx2[k]], tok[k], gsem[k]).wait()

  def wait_writeback(c, k):
    pltpu.make_async_copy(tok[k], out_hbm.at[pl.ds(w_base + c * CHUNK, CHUNK), 0],
                          wsem[k]).wait()

  def combine(c, k):
    base = w_base + c * CHUNK

    def add_body(g, _):
      xv = idx[k][pl.ds(g * LANES, LANES)]
      t = base + g * LANES + lax.iota(jnp.int32, LANES)
      pv = jnp.where(xv == 0, jnp.zeros((LANES,), jnp.int32),
                     lax.rem(t, L) + 1)
      for kk in range(LANES):
        r = g * LANES + kk
        p = pv[kk]
        for j in range(D // LANES):
          s = pl.ds(j * LANES, LANES)
          tok[k][r, s] = tok[k][r, s] + pos_l[p, s]
      return 0
    lax.fori_loop(0, CHUNK // LANES, add_body, 0)

    pltpu.async_copy(tok[k], out_hbm.at[pl.ds(base, CHUNK), 0], wsem[k])

  for c in range(LOOKAHEAD):
    fire_gather(c, c % NBUF)
  wait_gather(0)
  combine(0, 0)
  fire_gather(LOOKAHEAD, LOOKAHEAD % NBUF)

  def macro_body(i, _):
    c0 = 1 + i * NBUF
    for k in range(NBUF):
      c = c0 + k
      bc = (1 + k) % NBUF
      wait_gather(bc)
      combine(c, bc)
      bp = (1 + k + LOOKAHEAD) % NBUF
      wait_writeback(c - 1, bp)
      fire_gather(c + LOOKAHEAD, bp)
    return 0
  lax.fori_loop(0, STEADY, macro_body, 0)

  for c in range(1 + STEADY * NBUF, N_CHUNKS):
    wait_gather(c % NBUF)
    combine(c, c % NBUF)

  for c in range(N_CHUNKS - NBUF, N_CHUNKS):
    wait_writeback(c, c % NBUF)


@jax.jit
def kernel(x, token_table, pos_table):
  scratch = (
      [pltpu.VMEM((CHUNK,), jnp.int32) for _ in range(NBUF)]
      + [pltpu.VMEM((CHUNK,), jnp.int32) for _ in range(NBUF)]
      + [pltpu.VMEM((CHUNK, D), jnp.float32) for _ in range(NBUF)]
      + [pltpu.VMEM((L + 1, D), jnp.float32)]
      + [pltpu.SemaphoreType.DMA for _ in range(2 * NBUF)]
  )
  kfn = pl.kernel(
      _body,
      out_type=jax.ShapeDtypeStruct((N, 2, D), jnp.float32),
      mesh=plsc.VectorSubcoreMesh(core_axis_name="c", subcore_axis_name="s"),
      scratch_types=scratch,
      compiler_params=pltpu.CompilerParams(use_tc_tiling_on_sc=False),
  )
  tt = jnp.pad(token_table, ((0, 0), (0, DP - D))).reshape(2 * V, D)
  out = kfn(x.reshape(N), tt, pos_table)
  return out[:, 0, :].reshape(B, L, D)


# speedup vs baseline: 2.8770x; 2.8770x over previous
import jax
import jax.numpy as jnp
from jax import lax
from jax.experimental import pallas as pl
from jax.experimental.pallas import tpu as pltpu
from jax.experimental.pallas import tpu_sc as plsc

NC = 2
NS = 16
LANES = 16
NW = NC * NS

B, L, V, D = 4096, 200, 1000000, 64
DP = 128
N = B * L
N_PER_W = N // NW
CHUNK = 128
NBUF = 4
N_CHUNKS = N_PER_W // CHUNK
LOOKAHEAD = 3
STEADY = (N_CHUNKS - 1 - LOOKAHEAD) // NBUF


def _body(x_hbm, tok_hbm, pos_hbm, out_hbm, *refs):
  idx_all = refs[0]
  tok = refs[1:1 + NBUF]
  pos_l = refs[1 + NBUF]
  gsem = refs[2 + NBUF:2 + 2 * NBUF]
  wsem = refs[2 + 2 * NBUF:2 + 3 * NBUF]

  wid = lax.axis_index("s") * NC + lax.axis_index("c")
  w_base = wid * N_PER_W

  pltpu.sync_copy(pos_hbm, pos_l)
  pltpu.sync_copy(x_hbm.at[pl.ds(w_base, N_PER_W)], idx_all)

  def fire_gather(c, k):
    pltpu.async_copy(tok_hbm.at[idx_all.at[pl.ds(c * CHUNK, CHUNK)]], tok[k],
                     gsem[k])

  def wait_gather(c, k):
    pltpu.make_async_copy(tok_hbm.at[idx_all.at[pl.ds(c * CHUNK, CHUNK)]],
                          tok[k], gsem[k]).wait()

  def wait_writeback(c, k):
    pltpu.make_async_copy(tok[k], out_hbm.at[pl.ds(w_base + c * CHUNK, CHUNK)],
                          wsem[k]).wait()

  def combine(c, k):
    base = w_base + c * CHUNK

    def add_body(g, _):
      xv = idx_all[pl.ds(c * CHUNK + g * LANES, LANES)]
      t = base + g * LANES + lax.iota(jnp.int32, LANES)
      pv = jnp.where(xv == 0, jnp.zeros((LANES,), jnp.int32),
                     lax.rem(t, L) + 1)
      for kk in range(LANES):
        r = g * LANES + kk
        p = pv[kk]
        for j in range(D // LANES):
          s = pl.ds(j * LANES, LANES)
          tok[k][r, s] = tok[k][r, s] + pos_l[p, s]
      return 0
    lax.fori_loop(0, CHUNK // LANES, add_body, 0)

    pltpu.async_copy(tok[k], out_hbm.at[pl.ds(base, CHUNK)], wsem[k])

  for c in range(LOOKAHEAD):
    fire_gather(c, c % NBUF)
  wait_gather(0, 0)
  combine(0, 0)
  fire_gather(LOOKAHEAD, LOOKAHEAD % NBUF)

  def macro_body(i, _):
    c0 = 1 + i * NBUF
    for k in range(NBUF):
      c = c0 + k
      bc = (1 + k) % NBUF
      wait_gather(c, bc)
      combine(c, bc)
      bp = (1 + k + LOOKAHEAD) % NBUF
      wait_writeback(c - 1, bp)
      fire_gather(c + LOOKAHEAD, bp)
    return 0
  lax.fori_loop(0, STEADY, macro_body, 0)

  for c in range(1 + STEADY * NBUF, N_CHUNKS):
    wait_gather(c, c % NBUF)
    combine(c, c % NBUF)

  for c in range(N_CHUNKS - NBUF, N_CHUNKS):
    wait_writeback(c, c % NBUF)


@jax.jit
def kernel(x, token_table, pos_table):
  scratch = (
      [pltpu.VMEM((N_PER_W,), jnp.int32)]
      + [pltpu.VMEM((CHUNK, DP), jnp.float32) for _ in range(NBUF)]
      + [pltpu.VMEM((L + 1, D), jnp.float32)]
      + [pltpu.SemaphoreType.DMA for _ in range(2 * NBUF)]
  )
  kfn = pl.kernel(
      _body,
      out_type=jax.ShapeDtypeStruct((N, DP), jnp.float32),
      mesh=plsc.VectorSubcoreMesh(core_axis_name="c", subcore_axis_name="s"),
      scratch_types=scratch,
  )
  tt = jnp.pad(token_table, ((0, 0), (0, DP - D)))
  out = kfn(x.reshape(N), tt, pos_table)
  return out[:, :D].reshape(B, L, D)
